# Initial kernel scaffold; baseline (speedup 1.0000x reference)
import jax
import jax.numpy as jnp
from jax.experimental import pallas as pl


def kernel(timestamp, M_w, W_w, D_w, H_w):
    raise NotImplementedError("write your pallas kernel here")


# speedup vs baseline: 40.9837x; 40.9837x over previous
import functools

import jax
import jax.numpy as jnp
from jax import lax
from jax.experimental import pallas as pl
from jax.experimental.pallas import tpu as pltpu
from jax.experimental.pallas import tpu_sc as plsc

_SUP = (12.0, 53.0, 31.0, 23.0)
_INF = (1.0, 1.0, 1.0, 0.0)
_NC, _NS, _L = 2, 16, 16
_NW = _NC * _NS
_T = 256


@functools.lru_cache(maxsize=None)
def _build(n_rows):
    per_w = n_rows // _NW
    assert per_w * _NW == n_rows and per_w % _T == 0
    n_blk = per_w // _T

    def body(ts_hbm, tbl_hbm, out_hbm, ts_v, out_v, tbl_v,
             in_sem0, in_sem1, out_sem0, out_sem1):
        in_sems = (in_sem0, in_sem1)
        out_sems = (out_sem0, out_sem1)
        wid = lax.axis_index("s") * _NC + lax.axis_index("c")
        base = wid * per_w

        pltpu.sync_copy(tbl_hbm, tbl_v)
        ab = []
        for f in range(4):
            inv = 1.0 / (_INF[f] - _SUP[f])
            for h in range(2):
                e0 = tbl_v[pl.ds(64 * f + 16 * h, _L)]
                e1 = tbl_v[pl.ds(64 * f + 32 + 16 * h, _L)]
                ab.append(((e1 - e0) * inv,
                           (_SUP[f] * e0 - _INF[f] * e1) * inv))

        def in_copy(g, b):
            return pltpu.make_async_copy(
                ts_hbm.at[pl.ds((base + g * _T) * 5, _T * 5)],
                ts_v.at[b], in_sems[b])

        def out_copy(g, b):
            return pltpu.make_async_copy(
                out_v.at[b],
                out_hbm.at[pl.ds((base + g * _T) * 128, _T * 128)],
                out_sems[b])

        def compute(b):
            def tstep(j, carry):
                s = 80 * j
                chunks = [ts_v[b, pl.ds(s + _L * k, _L)] for k in range(5)]
                for r in range(16):
                    o = 128 * (16 * j + r)
                    for f in range(4):
                        p = 5 * r + f + 1
                        v = chunks[p // _L][p % _L]
                        for h in range(2):
                            a, c = ab[2 * f + h]
                            out_v[b, pl.ds(o + 32 * f + 16 * h, _L)] = v * a + c
                return carry
            lax.fori_loop(0, _T // 16, tstep, 0)

        in_copy(0, 0).start()
        in_copy(1, 1).start()

        def outer(i, carry):
            for b in range(2):
                g = 2 * i + b
                in_copy(g, b).wait()

                @pl.when(i >= 1)
                def _wait_out():
                    out_copy(g - 2, b).wait()

                compute(b)
                out_copy(g, b).start()

                @pl.when(g + 2 < n_blk)
                def _next_in():
                    in_copy(g + 2, b).start()
            return carry

        lax.fori_loop(0, n_blk // 2, outer, 0)
        out_copy(n_blk - 2, 0).wait()
        out_copy(n_blk - 1, 1).wait()

    return pl.kernel(
        body,
        out_type=jax.ShapeDtypeStruct((n_rows * 128,), jnp.float32),
        mesh=plsc.VectorSubcoreMesh(
            core_axis_name="c", subcore_axis_name="s",
            num_cores=_NC, num_subcores=_NS),
        scratch_types=[
            pltpu.VMEM((2, _T * 5), jnp.float32),
            pltpu.VMEM((2, _T * 128), jnp.float32),
            pltpu.VMEM((4 * 64,), jnp.float32),
            pltpu.SemaphoreType.DMA,
            pltpu.SemaphoreType.DMA,
            pltpu.SemaphoreType.DMA,
            pltpu.SemaphoreType.DMA,
        ],
    )


def kernel(timestamp, M_w, W_w, D_w, H_w):
    shape = timestamp.shape[:-1]
    n_rows = 1
    for d in shape:
        n_rows *= d
    ts = timestamp.reshape(n_rows * 5)
    tbl = jnp.concatenate([M_w.reshape(-1), W_w.reshape(-1),
                           D_w.reshape(-1), H_w.reshape(-1)])
    out = _build(n_rows)(ts, tbl)
    return out.reshape(*shape, 128)


# speedup vs baseline: 41.3202x; 1.0082x over previous
import functools

import jax
import jax.numpy as jnp
from jax import lax
from jax.experimental import pallas as pl
from jax.experimental.pallas import tpu as pltpu
from jax.experimental.pallas import tpu_sc as plsc

_SUP = (12.0, 53.0, 31.0, 23.0)
_INF = (1.0, 1.0, 1.0, 0.0)
_NC, _NS, _L = 2, 16, 16
_NW = _NC * _NS
_T = 256


@functools.lru_cache(maxsize=None)
def _build(n_rows):
    per_w = n_rows // _NW
    assert per_w * _NW == n_rows and per_w % _T == 0
    n_blk = per_w // _T

    def body(ts_hbm, tbl_hbm, out_hbm, ts_v, out_v, tbl_v,
             in_sem0, in_sem1, out_sem0, out_sem1):
        in_sems = (in_sem0, in_sem1)
        out_sems = (out_sem0, out_sem1)
        wid = lax.axis_index("s") * _NC + lax.axis_index("c")
        base = wid * per_w

        pltpu.sync_copy(tbl_hbm, tbl_v)
        ab = []
        for f in range(4):
            inv = 1.0 / (_INF[f] - _SUP[f])
            for h in range(2):
                e0 = tbl_v[pl.ds(64 * f + 16 * h, _L)]
                e1 = tbl_v[pl.ds(64 * f + 32 + 16 * h, _L)]
                ab.append(((e1 - e0) * inv,
                           (_SUP[f] * e0 - _INF[f] * e1) * inv))

        def in_copy(g, b):
            return pltpu.make_async_copy(
                ts_hbm.at[pl.ds((base + g * _T) * 5, _T * 5)],
                ts_v.at[b], in_sems[b])

        def out_copy(g, b):
            return pltpu.make_async_copy(
                out_v.at[b],
                out_hbm.at[pl.ds((base + g * _T) * 128, _T * 128)],
                out_sems[b])

        def compute(b):
            @plsc.parallel_loop(0, _T // 16, unroll=2)
            def tstep(j):
                s = 80 * j
                chunks = [ts_v[b, pl.ds(s + _L * k, _L)] for k in range(5)]
                for r in range(16):
                    o = 128 * (16 * j + r)
                    for f in range(4):
                        p = 5 * r + f + 1
                        v = chunks[p // _L][p % _L]
                        for h in range(2):
                            a, c = ab[2 * f + h]
                            out_v[b, pl.ds(o + 32 * f + 16 * h, _L)] = v * a + c

        in_copy(0, 0).start()
        in_copy(1, 1).start()

        def outer(i, carry):
            for b in range(2):
                g = 2 * i + b
                in_copy(g, b).wait()

                @pl.when(i >= 1)
                def _wait_out():
                    out_copy(g - 2, b).wait()

                compute(b)
                out_copy(g, b).start()

                @pl.when(g + 2 < n_blk)
                def _next_in():
                    in_copy(g + 2, b).start()
            return carry

        lax.fori_loop(0, n_blk // 2, outer, 0)
        out_copy(n_blk - 2, 0).wait()
        out_copy(n_blk - 1, 1).wait()

    return pl.kernel(
        body,
        out_type=jax.ShapeDtypeStruct((n_rows * 128,), jnp.float32),
        mesh=plsc.VectorSubcoreMesh(
            core_axis_name="c", subcore_axis_name="s",
            num_cores=_NC, num_subcores=_NS),
        scratch_types=[
            pltpu.VMEM((2, _T * 5), jnp.float32),
            pltpu.VMEM((2, _T * 128), jnp.float32),
            pltpu.VMEM((4 * 64,), jnp.float32),
            pltpu.SemaphoreType.DMA,
            pltpu.SemaphoreType.DMA,
            pltpu.SemaphoreType.DMA,
            pltpu.SemaphoreType.DMA,
        ],
    )


def kernel(timestamp, M_w, W_w, D_w, H_w):
    shape = timestamp.shape[:-1]
    n_rows = 1
    for d in shape:
        n_rows *= d
    ts = timestamp.reshape(n_rows * 5)
    tbl = jnp.concatenate([M_w.reshape(-1), W_w.reshape(-1),
                           D_w.reshape(-1), H_w.reshape(-1)])
    out = _build(n_rows)(ts, tbl)
    return out.reshape(*shape, 128)


# speedup vs baseline: 91.1017x; 2.2048x over previous
import functools

import jax
import jax.numpy as jnp
from jax import lax
from jax.experimental import pallas as pl
from jax.experimental.pallas import tpu as pltpu
from jax.experimental.pallas import tpu_sc as plsc

_SUP = (12.0, 53.0, 31.0, 23.0)
_INF = (1.0, 1.0, 1.0, 0.0)
_NC, _NS, _L = 2, 16, 16
_NW = _NC * _NS
_JB = 4
_IB = 32


@functools.lru_cache(maxsize=None)
def _build(n_i, n_j):
    slab = n_i // _NW
    n_jc = n_j // _JB
    n_sub = slab // _IB
    assert slab * _NW == n_i and n_jc * _JB == n_j and n_sub * _IB == slab
    assert n_jc % 2 == 0 and n_sub % 2 == 0 and slab % 128 == 0

    def body(ts_hbm, tbl_hbm, out_hbm, ts_v, out_v, tbl_v,
             in_sem0, in_sem1, out_sem0, out_sem1):
        in_sems = (in_sem0, in_sem1)
        out_sems = (out_sem0, out_sem1)
        wid = lax.axis_index("s") * _NC + lax.axis_index("c")
        i0 = wid * slab

        pltpu.sync_copy(tbl_hbm, tbl_v)
        ab = []
        for f in range(4):
            inv = 1.0 / (_INF[f] - _SUP[f])
            for h in range(2):
                e0 = tbl_v[pl.ds(64 * f + 16 * h, _L)]
                e1 = tbl_v[pl.ds(64 * f + 32 + 16 * h, _L)]
                ab.append(((e1 - e0) * inv,
                           (_SUP[f] * e0 - _INF[f] * e1) * inv))

        def in_copy(jc, b, d):
            return pltpu.make_async_copy(
                ts_hbm.at[d + 1, pl.ds(jc * _JB, _JB), pl.ds(i0, slab)],
                ts_v.at[b, d], in_sems[b])

        def out_copy(jc, k, b2):
            return pltpu.make_async_copy(
                out_v.at[b2],
                out_hbm.at[pl.ds(i0 + k * _IB, _IB),
                           pl.ds(jc * _JB, _JB), pl.ds(0, 128)],
                out_sems[b2])

        def compute(jc, k, b, b2):
            @plsc.parallel_loop(0, _JB * (_IB // 16), unroll=1)
            def group(gi):
                jj = gi >> 1
                g = gi & 1
                vecs = [ts_v[b, d, jj, pl.ds(k * _IB + g * 16, _L)]
                        for d in range(4)]
                for ii in range(16):
                    row = g * 16 + ii
                    for f in range(4):
                        v = vecs[f][ii]
                        for h in range(2):
                            a, c = ab[2 * f + h]
                            out_v[b2, row, jj, pl.ds(32 * f + 16 * h, _L)] = (
                                v * a + c)

        for d in range(4):
            in_copy(0, 0, d).start()

        def jchunk(jc2, carry):
            for b in range(2):
                jc = jc2 * 2 + b
                for d in range(4):
                    in_copy(jc, b, d).wait()

                @pl.when(jc + 1 < n_jc)
                def _next_in():
                    for d in range(4):
                        in_copy(jc + 1, 1 - b, d).start()

                def sub(k2, carry2):
                    for b2 in range(2):
                        k = k2 * 2 + b2
                        s = jc * n_sub + k

                        @pl.when(s >= 2)
                        def _wait_out():
                            out_copy(jc, k, b2).wait()

                        compute(jc, k, b, b2)
                        out_copy(jc, k, b2).start()
                    return carry2
                lax.fori_loop(0, n_sub // 2, sub, 0)
            return carry
        lax.fori_loop(0, n_jc // 2, jchunk, 0)

        out_copy(n_jc - 1, n_sub - 2, 0).wait()
        out_copy(n_jc - 1, n_sub - 1, 1).wait()

    return pl.kernel(
        body,
        out_type=jax.ShapeDtypeStruct((n_i, n_j, 128), jnp.float32),
        mesh=plsc.VectorSubcoreMesh(
            core_axis_name="c", subcore_axis_name="s",
            num_cores=_NC, num_subcores=_NS),
        scratch_types=[
            pltpu.VMEM((2, 4, _JB, slab), jnp.float32),
            pltpu.VMEM((2, _IB, _JB, 128), jnp.float32),
            pltpu.VMEM((4 * 64,), jnp.float32),
            pltpu.SemaphoreType.DMA,
            pltpu.SemaphoreType.DMA,
            pltpu.SemaphoreType.DMA,
            pltpu.SemaphoreType.DMA,
        ],
    )


def kernel(timestamp, M_w, W_w, D_w, H_w):
    shape = timestamp.shape[:-1]
    n_rows = 1
    for d in shape:
        n_rows *= d
    n_j = timestamp.shape[-2]
    n_i = n_rows // n_j
    tsT = jnp.transpose(timestamp.reshape(n_i, n_j, 5), (2, 1, 0))
    tbl = jnp.concatenate([M_w.reshape(-1), W_w.reshape(-1),
                           D_w.reshape(-1), H_w.reshape(-1)])
    out = _build(n_i, n_j)(tsT, tbl)
    return out.reshape(*shape, 128)
